# Initial kernel scaffold; baseline (speedup 1.0000x reference)
import functools

import jax
import jax.numpy as jnp
from jax import lax
from jax.experimental import pallas as pl
from jax.experimental.pallas import tpu as pltpu
from jax.experimental.pallas import tpu_sc as plsc

T = 200
D = 32
B = 4096
NC = 2
NS = 16
NW = NC * NS
BPW = B // NW

CHUNK0 = 128
CHUNK1 = T - CHUNK0


def _sc_body(idx_hbm, table_hbm, pos_hbm, out_hbm, idx_v, rows_v, pos_v, sem):
    wid = lax.axis_index("s") * NC + lax.axis_index("c")
    pltpu.sync_copy(pos_hbm, pos_v)

    def row_body(i, carry):
        r = wid * BPW + i
        pltpu.sync_copy(idx_hbm.at[r], idx_v)
        cp0 = pltpu.async_copy(
            table_hbm.at[idx_v.at[pl.ds(0, CHUNK0)]],
            rows_v.at[pl.ds(0, CHUNK0)],
            sem,
        )
        cp1 = pltpu.async_copy(
            table_hbm.at[idx_v.at[pl.ds(CHUNK0, CHUNK1)]],
            rows_v.at[pl.ds(CHUNK0, CHUNK1)],
            sem,
        )
        cp0.wait()
        cp1.wait()

        def add_body(j, c):
            rows_v[j, pl.ds(0, 16)] = rows_v[j, pl.ds(0, 16)] + pos_v[j, pl.ds(0, 16)]
            rows_v[j, pl.ds(16, 16)] = rows_v[j, pl.ds(16, 16)] + pos_v[j, pl.ds(16, 16)]
            return c

        lax.fori_loop(0, T, add_body, 0)
        pltpu.sync_copy(rows_v, out_hbm.at[r])
        return carry

    lax.fori_loop(0, BPW, row_body, 0)


@jax.jit
def _seq_embed(idx, token_table, pos_table):
    mesh = plsc.VectorSubcoreMesh(core_axis_name="c", subcore_axis_name="s")
    fn = functools.partial(
        pl.kernel,
        mesh=mesh,
        out_type=jax.ShapeDtypeStruct((B, T, D), jnp.float32),
        scratch_types=[
            pltpu.VMEM((T,), jnp.int32),
            pltpu.VMEM((T, D), jnp.float32),
            pltpu.VMEM((T, D), jnp.float32),
            pltpu.SemaphoreType.DMA,
        ],
    )(_sc_body)
    return fn(idx, token_table, pos_table)


def kernel(inputs, token_table, pos_table):
    idx = inputs.astype(jnp.int32)
    return _seq_embed(idx, token_table, pos_table)


# speedup vs baseline: 1.1807x; 1.1807x over previous
import functools

import jax
import jax.numpy as jnp
from jax import lax
from jax.experimental import pallas as pl
from jax.experimental.pallas import tpu as pltpu
from jax.experimental.pallas import tpu_sc as plsc

T = 200
D = 32
B = 4096
NC = 2
NS = 16
NW = NC * NS
BPW = B // NW

CHUNK0 = 128
CHUNK1 = T - CHUNK0


def _sc_body(idx_hbm, table_hbm, pos_hbm, out_hbm, idx_v, rows_v, pos_v, sem):
    wid = lax.axis_index("s") * NC + lax.axis_index("c")
    pltpu.sync_copy(pos_hbm, pos_v)

    def row_body(i, carry):
        r = wid * BPW + i
        pltpu.sync_copy(idx_hbm.at[r], idx_v)
        cp0 = pltpu.async_copy(
            table_hbm.at[idx_v.at[pl.ds(0, CHUNK0)]],
            rows_v.at[pl.ds(0, CHUNK0)],
            sem,
        )
        cp1 = pltpu.async_copy(
            table_hbm.at[idx_v.at[pl.ds(CHUNK0, CHUNK1)]],
            rows_v.at[pl.ds(CHUNK0, CHUNK1)],
            sem,
        )
        cp0.wait()
        cp1.wait()

        def add_body(j, c):
            rows_v[j, pl.ds(0, 16)] = rows_v[j, pl.ds(0, 16)] + pos_v[j, pl.ds(0, 16)]
            rows_v[j, pl.ds(16, 16)] = rows_v[j, pl.ds(16, 16)] + pos_v[j, pl.ds(16, 16)]
            return c

        lax.fori_loop(0, T, add_body, 0)
        pltpu.sync_copy(rows_v, out_hbm.at[r])
        return carry

    lax.fori_loop(0, BPW, row_body, 0)


@jax.jit
def _seq_embed(idx, token_table, pos_table):
    mesh = plsc.VectorSubcoreMesh(core_axis_name="c", subcore_axis_name="s")
    fn = functools.partial(
        pl.kernel,
        mesh=mesh,
        out_type=jax.ShapeDtypeStruct((B, T, D), jnp.float32),
        scratch_types=[
            pltpu.VMEM((T,), jnp.int32),
            pltpu.VMEM((T, D), jnp.float32),
            pltpu.VMEM((T, D), jnp.float32),
            pltpu.SemaphoreType.DMA,
        ],
        compiler_params=pltpu.CompilerParams(use_tc_tiling_on_sc=False),
    )(_sc_body)
    return fn(idx, token_table, pos_table)


def kernel(inputs, token_table, pos_table):
    idx = inputs.astype(jnp.int32)
    return _seq_embed(idx, token_table, pos_table)


# speedup vs baseline: 1.4809x; 1.2543x over previous
import functools

import jax
import jax.numpy as jnp
from jax import lax
from jax.experimental import pallas as pl
from jax.experimental.pallas import tpu as pltpu
from jax.experimental.pallas import tpu_sc as plsc

T = 200
D = 32
B = 4096
NC = 2
NS = 16
NW = NC * NS
BPW = B // NW
IPW = BPW * T

C = 2
ROWS = C * T
STEPS = BPW // C
NBUF = 4
LEAD = 2
GROUPS = STEPS // NBUF

CHUNKS = (128, 128, 128, 16)


def _sc_body(idx_hbm, table_hbm, pos_hbm, out_hbm, idx_v, r0, r1, r2, r3,
             pos_v, g0, g1, g2, g3, o0, o1, o2, o3):
    rows = (r0, r1, r2, r3)
    gsem = (g0, g1, g2, g3)
    osem = (o0, o1, o2, o3)
    wid = lax.axis_index("s") * NC + lax.axis_index("c")
    ibase = wid * IPW

    pltpu.sync_copy(pos_hbm, pos_v)
    pltpu.sync_copy(idx_hbm.at[pl.ds(ibase, IPW)], idx_v)

    def gather_copies(step, slot):
        cps = []
        off = 0
        for sz in CHUNKS:
            cps.append(pltpu.make_async_copy(
                table_hbm.at[idx_v.at[pl.ds(step * ROWS + off, sz)]],
                rows[slot].at[pl.ds(off, sz)],
                gsem[slot],
            ))
            off += sz
        return cps

    def out_copy(step, slot):
        return pltpu.make_async_copy(
            rows[slot],
            out_hbm.at[pl.ds(ibase + step * ROWS, ROWS)],
            osem[slot],
        )

    for s in range(LEAD):
        for cp in gather_copies(s, s):
            cp.start()

    def group_body(g, carry):
        for b in range(NBUF):
            s = g * NBUF + b
            for cp in gather_copies(s, b):
                cp.wait()

            def add_body(j, c):
                p0 = pos_v[j, pl.ds(0, 16)]
                p1 = pos_v[j, pl.ds(16, 16)]
                rv = rows[b]
                for r in range(C):
                    k = r * T + j
                    rv[k, pl.ds(0, 16)] = rv[k, pl.ds(0, 16)] + p0
                    rv[k, pl.ds(16, 16)] = rv[k, pl.ds(16, 16)] + p1
                return c

            lax.fori_loop(0, T, add_body, 0)
            out_copy(s, b).start()

            t = s + LEAD
            b2 = (b + LEAD) % NBUF
            if b < NBUF - LEAD:
                @pl.when(g >= 1)
                def _():
                    out_copy(t - NBUF, b2).wait()

                for cp in gather_copies(t, b2):
                    cp.start()
            else:
                @pl.when(g < GROUPS - 1)
                def _():
                    out_copy(t - NBUF, b2).wait()
                    for cp in gather_copies(t, b2):
                        cp.start()
        return carry

    lax.fori_loop(0, GROUPS, group_body, 0)

    for b in range(NBUF):
        out_copy(STEPS - NBUF + b, b).wait()


@jax.jit
def _seq_embed(idx, token_table, pos_table):
    mesh = plsc.VectorSubcoreMesh(core_axis_name="c", subcore_axis_name="s")
    fn = functools.partial(
        pl.kernel,
        mesh=mesh,
        out_type=jax.ShapeDtypeStruct((B * T, D), jnp.float32),
        scratch_types=[
            pltpu.VMEM((IPW,), jnp.int32),
            pltpu.VMEM((ROWS, D), jnp.float32),
            pltpu.VMEM((ROWS, D), jnp.float32),
            pltpu.VMEM((ROWS, D), jnp.float32),
            pltpu.VMEM((ROWS, D), jnp.float32),
            pltpu.VMEM((T, D), jnp.float32),
        ] + [pltpu.SemaphoreType.DMA] * 8,
        compiler_params=pltpu.CompilerParams(use_tc_tiling_on_sc=False),
    )(_sc_body)
    return fn(idx, token_table, pos_table)


def kernel(inputs, token_table, pos_table):
    idx = inputs.reshape(-1).astype(jnp.int32)
    return _seq_embed(idx, token_table, pos_table).reshape(B, T, D)


# speedup vs baseline: 2.5465x; 1.7195x over previous
import functools

import jax
import jax.numpy as jnp
from jax import lax
from jax.experimental import pallas as pl
from jax.experimental.pallas import tpu as pltpu
from jax.experimental.pallas import tpu_sc as plsc

T = 200
D = 32
B = 4096
NC = 2
NS = 16
NW = NC * NS
LPW = B // NW

NBUF = 10
LEAD = 7
GROUPS = T // NBUF
DG = D // 8
PITCH = LPW + 1


def _sc_body(idx_hbm, table_hbm, pos_hbm, out_hbm, idx_v, pos_v, rows, outb,
             gsem, osem):
    wid = lax.axis_index("s") * NC + lax.axis_index("c")

    pltpu.sync_copy(pos_hbm, pos_v)
    pltpu.sync_copy(idx_hbm.at[:, wid], idx_v)

    def gather_copy(step, slot):
        return pltpu.make_async_copy(
            table_hbm.at[idx_v.at[step // 8, step % 8]],
            rows[slot],
            gsem[slot],
        )

    def out_copies(step, slot):
        return [
            pltpu.make_async_copy(
                outb[slot].at[:, :, pl.ds(0, LPW)],
                out_hbm.at[step, :, wid],
                osem[slot],
            )
        ]

    for s in range(LEAD):
        gather_copy(s, s).start()

    iota16 = lax.iota(jnp.int32, 16)
    dg_lo = iota16 // 8
    ds_lo = iota16 % 8
    dg_hi = dg_lo + 2

    def group_body(g, carry):
        for bslot in range(NBUF):
            t = g * NBUF + bslot
            gather_copy(t, bslot).wait()

            rv = rows[bslot]
            ov = outb[bslot]
            p0 = pos_v[t, pl.ds(0, 16)]
            p1 = pos_v[t, pl.ds(16, 16)]

            @plsc.parallel_loop(0, LPW, unroll=8)
            def _(r):
                rvec = jnp.full((16,), r, jnp.int32)
                v0 = rv[r, pl.ds(0, 16)] + p0
                v1 = rv[r, pl.ds(16, 16)] + p1
                plsc.store_scatter(ov, [dg_lo, ds_lo, rvec], v0)
                plsc.store_scatter(ov, [dg_hi, ds_lo, rvec], v1)

            for cp in out_copies(t, bslot):
                cp.start()

            b2 = (bslot + LEAD) % NBUF
            if bslot < NBUF - LEAD:
                @pl.when(g >= 1)
                def _():
                    for cp in out_copies(t + LEAD - NBUF, b2):
                        cp.wait()

                gather_copy(t + LEAD, b2).start()
            else:
                @pl.when(g < GROUPS - 1)
                def _():
                    for cp in out_copies(t + LEAD - NBUF, b2):
                        cp.wait()
                    gather_copy(t + LEAD, b2).start()
        return carry

    lax.fori_loop(0, GROUPS, group_body, 0)

    for bslot in range(NBUF):
        for cp in out_copies(T - NBUF + bslot, bslot):
            cp.wait()


def _sc_entry(idx_hbm, table_hbm, pos_hbm, out_hbm, *scratch):
    idx_v = scratch[0]
    pos_v = scratch[1]
    rows = scratch[2:2 + NBUF]
    outb = scratch[2 + NBUF:2 + 2 * NBUF]
    gsem = scratch[2 + 2 * NBUF:2 + 3 * NBUF]
    osem = scratch[2 + 3 * NBUF:2 + 4 * NBUF]
    _sc_body(idx_hbm, table_hbm, pos_hbm, out_hbm, idx_v, pos_v, rows, outb,
             gsem, osem)


@jax.jit
def _seq_embed(idx4d, token_table, pos_table):
    mesh = plsc.VectorSubcoreMesh(core_axis_name="c", subcore_axis_name="s")
    fn = functools.partial(
        pl.kernel,
        mesh=mesh,
        out_type=jax.ShapeDtypeStruct((T, DG, NW, 8, LPW), jnp.float32),
        scratch_types=[
            pltpu.VMEM((T // 8, 8, LPW), jnp.int32),
            pltpu.VMEM((T, D), jnp.float32),
        ] + [pltpu.VMEM((LPW, D), jnp.float32)] * NBUF
          + [pltpu.VMEM((DG, 8, PITCH), jnp.float32)] * NBUF
          + [pltpu.SemaphoreType.DMA] * (2 * NBUF),
        compiler_params=pltpu.CompilerParams(
            use_tc_tiling_on_sc=False, needs_layout_passes=False,
            disable_bounds_checks=True),
    )(_sc_entry)
    return fn(idx4d, token_table, pos_table)


def kernel(inputs, token_table, pos_table):
    idx = inputs.astype(jnp.int32)
    idx4d = idx.T.reshape(T // 8, 8, NW, LPW).transpose(0, 2, 1, 3)
    out5d = _seq_embed(idx4d, token_table, pos_table)
    return out5d.transpose(2, 4, 0, 1, 3).reshape(B, T, D)
